# baseline (speedup 1.0000x reference)
import functools

import jax
import jax.numpy as jnp
from jax import lax
from jax.experimental import pallas as pl
from jax.experimental.pallas import tpu as pltpu
from jax.experimental.pallas import tpu_sc as plsc

B = 16384
V = 100000
FEAT = 16
H = 64
D = H // 2

_OC = 8192


def _gather_sc(table_t, idx):
    info = plsc.get_sparse_core_info()
    nw = info.num_cores * info.num_subcores
    n_oc = B // _OC
    mesh = plsc.VectorSubcoreMesh(core_axis_name="c", subcore_axis_name="s")

    @functools.partial(
        pl.kernel,
        mesh=mesh,
        compiler_params=pltpu.CompilerParams(needs_layout_passes=False),
        out_type=jax.ShapeDtypeStruct((D, B), jnp.float32),
        scratch_types=[
            pltpu.VMEM((V,), jnp.float32),
            pltpu.VMEM((B,), jnp.int32),
            pltpu.VMEM((_OC,), jnp.float32),
            pltpu.SemaphoreType.DMA,
        ],
    )
    def k(tab_hbm, idx_hbm, out_hbm, row_v, idx_v, oc_v, sem):
        wid = (lax.axis_index("c") * info.num_subcores + lax.axis_index("s"))
        row_cp = pltpu.async_copy(tab_hbm.at[wid], row_v, sem)
        pltpu.sync_copy(idx_hbm, idx_v)
        row_cp.wait()
        UNROLL = 16
        STEP = 16 * UNROLL

        def chunk(cc, carry):
            def body(i, c2):
                for u in range(UNROLL):
                    iv = idx_v[pl.ds(cc * _OC + i * STEP + u * 16, 16)]
                    oc_v[pl.ds(i * STEP + u * 16, 16)] = (
                        plsc.load_gather(row_v, [iv]))
                return c2
            lax.fori_loop(0, _OC // STEP, body, carry)
            pltpu.sync_copy(oc_v, out_hbm.at[wid, pl.ds(cc * _OC, _OC)])
            return carry

        lax.fori_loop(0, n_oc, chunk, 0)

    return k(table_t, idx)


def _dense_body(te_ref, cf_ref, wf_ref, bf_ref, wm1_ref, wm2_ref, bm_ref,
                g_ref, bb_ref, out_ref):
    te = te_ref[...]
    cf = cf_ref[...]
    dn = (((1,), (0,)), ((), ()))
    feat = lax.dot_general(wf_ref[...], cf, dn,
                           preferred_element_type=jnp.float32) + bf_ref[...]
    h = (lax.dot_general(wm1_ref[...], te, dn,
                         preferred_element_type=jnp.float32)
         + lax.dot_general(wm2_ref[...], feat, dn,
                           preferred_element_type=jnp.float32)
         + bm_ref[...])
    mu = jnp.mean(h, axis=0, keepdims=True)
    d = h - mu
    var = jnp.mean(d * d, axis=0, keepdims=True)
    hn = d * lax.rsqrt(var + 1e-5) * g_ref[...] + bb_ref[...]
    out_ref[...] = jnp.maximum(hn, 0.0)


def _dense_tc(te_t, cf_t, W_feat, b_feat, W_merge, b_merge, ln_gamma,
              ln_beta):
    RC = 4096
    grid = (B // RC,)
    col_spec = lambda h: pl.BlockSpec((h, RC), lambda i: (0, i))
    rep = lambda shape: pl.BlockSpec(shape, lambda i: (0, 0))
    return pl.pallas_call(
        _dense_body,
        grid=grid,
        in_specs=[
            col_spec(D),
            col_spec(FEAT),
            rep((D, FEAT)),
            rep((D, 1)),
            rep((H, D)),
            rep((H, D)),
            rep((H, 1)),
            rep((H, 1)),
            rep((H, 1)),
        ],
        out_specs=col_spec(H),
        out_shape=jax.ShapeDtypeStruct((H, B), jnp.float32),
    )(te_t, cf_t, W_feat, b_feat.reshape(D, 1),
      W_merge[:, :D], W_merge[:, D:], b_merge.reshape(H, 1),
      ln_gamma.reshape(H, 1), ln_beta.reshape(H, 1))


def kernel(cell_types, cell_features, embed_table, W_feat, b_feat, W_merge,
           b_merge, ln_gamma, ln_beta):
    te_t = _gather_sc(embed_table.T, cell_types.astype(jnp.int32))
    h_t = _dense_tc(te_t, cell_features.T, W_feat, b_feat, W_merge,
                    b_merge, ln_gamma, ln_beta)
    return h_t.T
